# Initial kernel scaffold; baseline (speedup 1.0000x reference)
import jax
import jax.numpy as jnp
from jax.experimental import pallas as pl


def kernel(enc_outputs, calculate_memory_context, memory_fingerprint, memory_context, weight, k):
    raise NotImplementedError("write your pallas kernel here")


# speedup vs baseline: 1.5837x; 1.5837x over previous
import functools

import jax
import jax.numpy as jnp
from jax import lax
from jax.experimental import pallas as pl
from jax.experimental.pallas import tpu as pltpu
from jax.experimental.pallas import tpu_sc as plsc

B = 32
D = 512
M = 1576
GROUP = 64
K = 50
KPAD = 64
NC = 2
NS = 16
EPS = 1e-8


def _tc_body(q_ref, mc_ref, w_ref, ridx_ref, wb_ref):
    q = q_ref[...]
    mc = mc_ref[...]
    qn = jnp.maximum(jnp.sqrt(jnp.sum(q * q, axis=1, keepdims=True)), EPS)
    mn = jnp.maximum(jnp.sqrt(jnp.sum(mc * mc, axis=1, keepdims=True)), EPS)
    cos = lax.dot_general(
        q / qn, mc / mn,
        dimension_numbers=(((1,), (1,)), ((), ())),
        preferred_element_type=jnp.float32,
        precision=lax.Precision.HIGHEST,
    )

    col = lax.broadcasted_iota(jnp.int32, (B, M), 1)
    vals = []
    idxs = []
    for _ in range(K):
        mx = jnp.max(cos, axis=1, keepdims=True)
        am = jnp.min(jnp.where(cos >= mx, col, M), axis=1, keepdims=True)
        vals.append(mx)
        idxs.append(am)
        cos = jnp.where(col == am, -jnp.inf, cos)
    v = jnp.concatenate(vals, axis=1)
    ii = jnp.concatenate(idxs, axis=1)

    e = jnp.exp(v - v[:, 0:1])
    sm = e / jnp.sum(e, axis=1, keepdims=True)

    w = w_ref[0]
    cols = jnp.concatenate(
        [sm * w,
         jnp.full((B, 1), 1.0, dtype=jnp.float32) - w,
         jnp.zeros((B, KPAD - K - 1), dtype=jnp.float32)],
        axis=1)
    wb_ref[...] = jnp.broadcast_to(cols[:, :, None], (B, KPAD, 16))

    g = lax.broadcasted_iota(jnp.int32, (B, K, GROUP), 2)
    ridx_ref[...] = ii[:, :, None] * GROUP + g


def _tc_stage(q, mc, w):
    return pl.pallas_call(
        _tc_body,
        out_shape=[
            jax.ShapeDtypeStruct((B, K, GROUP), jnp.int32),
            jax.ShapeDtypeStruct((B, KPAD, 16), jnp.float32),
        ],
        in_specs=[
            pl.BlockSpec(memory_space=pltpu.VMEM),
            pl.BlockSpec(memory_space=pltpu.VMEM),
            pl.BlockSpec(memory_space=pltpu.SMEM),
        ],
    )(q, mc, w)


_SC_MESH = plsc.VectorSubcoreMesh(core_axis_name="c", subcore_axis_name="s",
                                  num_cores=NC, num_subcores=NS)


@functools.partial(
    pl.kernel,
    out_type=jax.ShapeDtypeStruct((B, GROUP, D), jnp.float32),
    mesh=_SC_MESH,
    scratch_types=[
        pltpu.VMEM((K, GROUP), jnp.int32),
        pltpu.VMEM((KPAD * 16,), jnp.float32),
        pltpu.VMEM((GROUP, D), jnp.float32),
        pltpu.VMEM((GROUP, D), jnp.float32),
        pltpu.VMEM((GROUP, D), jnp.float32),
        pltpu.SemaphoreType.DMA,
        pltpu.SemaphoreType.DMA,
    ],
)
def _sc_stage(fp_hbm, ridx_hbm, wb_hbm, enc_hbm, out_hbm,
              ridx_v, wb_v, acc_v, buf0, buf1, sem0, sem1):
    b = lax.axis_index("s") * NC + lax.axis_index("c")

    pltpu.sync_copy(ridx_hbm.at[b], ridx_v)
    pltpu.sync_copy(wb_hbm.at[b], wb_v)

    pltpu.sync_copy(enc_hbm.at[b], buf0)
    w_enc = wb_v[pl.ds(K * 16, 16)]

    def init_row(r, _):
        for kk in range(D // 16):
            acc_v[r, pl.ds(kk * 16, 16)] = buf0[r, pl.ds(kk * 16, 16)] * w_enc
        return 0
    lax.fori_loop(0, GROUP, init_row, 0)

    def start_gather(j, buf, sem):
        pltpu.async_copy(fp_hbm.at[ridx_v.at[j]], buf, sem)

    def wait_gather(buf, sem):
        pltpu.make_async_copy(fp_hbm.at[ridx_v.at[0]], buf, sem).wait()

    def accumulate(buf, j):
        wsplat = wb_v[pl.ds(j * 16, 16)]

        def rowf(r, _):
            for kk in range(D // 16):
                plsc.addupdate(acc_v.at[r, pl.ds(kk * 16, 16)],
                               buf[r, pl.ds(kk * 16, 16)] * wsplat)
            return 0
        lax.fori_loop(0, GROUP, rowf, 0)

    start_gather(0, buf0, sem0)
    start_gather(1, buf1, sem1)

    def body(jj, _):
        j0 = 2 * jj
        wait_gather(buf0, sem0)
        accumulate(buf0, j0)

        @pl.when(jj < K // 2 - 1)
        def _():
            start_gather(j0 + 2, buf0, sem0)

        wait_gather(buf1, sem1)
        accumulate(buf1, j0 + 1)

        @pl.when(jj < K // 2 - 1)
        def _():
            start_gather(j0 + 3, buf1, sem1)
        return 0
    lax.fori_loop(0, K // 2, body, 0)

    pltpu.sync_copy(acc_v, out_hbm.at[b])


def kernel(enc_outputs, calculate_memory_context, memory_fingerprint,
           memory_context, weight, k):
    del k
    ridx, wb = _tc_stage(calculate_memory_context, memory_context, weight)
    out = _sc_stage(memory_fingerprint, ridx,
                    wb.reshape(B, KPAD * 16), enc_outputs)
    return out
